# baseline (speedup 1.0000x reference)
import functools

import jax
import jax.numpy as jnp
from jax import lax
from jax.experimental import pallas as pl
from jax.experimental.pallas import tpu as pltpu
from jax.experimental.pallas import tpu_sc as plsc

_NC = 2
_NS = 16
_L = 16

_B0 = 48
_S = 4
_R = 128


def _pool_tc_body(x_ref, o_ref, *, inv_n):
    o_ref[...] = jnp.sum(x_ref[...], axis=1, keepdims=True) * inv_n


def _sc_pool_body(x_hbm, out_hbm, buf0, buf1, buf2, stage, sem0, sem1, sem2, *,
                  b0, bsc, n, c, s_parts, r_chunk):
    cid = lax.axis_index("c")
    sid = lax.axis_index("s")
    per_core = bsc // _NC
    tasks_per_core = per_core * s_parts
    tpw = tasks_per_core // _NS
    rows_per_task = n // s_parts
    nchunks = rows_per_task // r_chunk
    bufs = (buf0, buf1, buf2)
    sems = (sem0, sem1, sem2)
    nlane = c // _L

    for t in range(tpw):
        task = sid * tpw + t
        slot = task // s_parts
        part = task % s_parts
        gbatch = b0 + cid * per_core + slot
        r0 = part * rows_per_task

        def _start(k):
            return pltpu.async_copy(
                x_hbm.at[gbatch, pl.ds(r0 + k * r_chunk, r_chunk), :],
                bufs[k % 3], sems[k % 3])

        unroll = 8

        def _mk_body(bref):
            def body(g, a):
                for dr in range(unroll):
                    a = tuple(a[l] + bref[g * unroll + dr, pl.ds(l * _L, _L)]
                              for l in range(nlane))
                return a
            return body

        acc = tuple(jnp.zeros((_L,), jnp.float32) for _ in range(nlane))
        cps = [None] * (nchunks + 1)
        cps[0] = _start(0)
        cps[1] = _start(1)
        for k in range(nchunks):
            if k + 2 < nchunks:
                cps[k + 2] = _start(k + 2)
            cps[k].wait()
            acc = lax.fori_loop(0, r_chunk // unroll,
                                _mk_body(bufs[k % 3]), acc)

        for l in range(nlane):
            stage[pl.ds(l * _L, _L)] = acc[l]
        gtask = cid * tasks_per_core + task
        pltpu.sync_copy(stage, out_hbm.at[gtask])


def _sim_body(ptc_ref, psc_ref, agg_ref, *, b, b0, s_parts, inv_n):
    bsc = b - b0
    p_tc = ptc_ref[...].reshape(b0, -1)
    part = psc_ref[...]
    g = bsc * s_parts
    per_core = bsc // _NC
    tasks_per_core = per_core * s_parts
    bi = lax.broadcasted_iota(jnp.int32, (bsc, g), 0)
    ti = lax.broadcasted_iota(jnp.int32, (bsc, g), 1)
    owner = (ti // tasks_per_core) * per_core + (ti % tasks_per_core) // s_parts
    m = jnp.where(owner == bi, 1.0, 0.0).astype(jnp.float32)
    p_sc = jnp.dot(m, part, preferred_element_type=jnp.float32) * inv_n
    p = jnp.concatenate([p_tc, p_sc], axis=0)

    s2 = jnp.sum(p * p, axis=1, keepdims=True)
    norm = jnp.sqrt(s2)
    xn = p / jnp.maximum(norm, 1e-12)
    s = jnp.dot(xn, xn.T, preferred_element_type=jnp.float32)
    rows = lax.broadcasted_iota(jnp.int32, (b, b), 0)
    cols = lax.broadcasted_iota(jnp.int32, (b, b), 1)
    s = jnp.where(rows == cols, 0.0, s)
    maxv = jnp.max(s, axis=1, keepdims=True)
    cand = jnp.where(s == maxv, cols, b)
    amin = jnp.min(cand, axis=1, keepdims=True)
    attn = jnp.where(cols == amin, maxv, 0.0)
    agg_ref[...] = jnp.dot(attn, p, preferred_element_type=jnp.float32)


def _add_body(x_ref, agg_ref, o_ref):
    o_ref[...] = x_ref[...] + agg_ref[...]


def kernel(x):
    b, n, c = x.shape
    b0 = _B0
    bsc = b - b0

    pool_tc = pl.pallas_call(
        functools.partial(_pool_tc_body, inv_n=1.0 / n),
        grid=(b0,),
        in_specs=[pl.BlockSpec((1, n, c), lambda i: (i, 0, 0))],
        out_specs=pl.BlockSpec((1, 1, c), lambda i: (i, 0, 0)),
        out_shape=jax.ShapeDtypeStruct((b0, 1, c), jnp.float32),
    )(x)

    mesh = plsc.VectorSubcoreMesh(
        core_axis_name="c", subcore_axis_name="s",
        num_cores=_NC, num_subcores=_NS)
    part_sc = pl.kernel(
        functools.partial(_sc_pool_body, b0=b0, bsc=bsc, n=n, c=c,
                          s_parts=_S, r_chunk=_R),
        out_type=jax.ShapeDtypeStruct((bsc * _S, c), jnp.float32),
        mesh=mesh,
        scratch_types=[
            pltpu.VMEM((_R, c), jnp.float32),
            pltpu.VMEM((_R, c), jnp.float32),
            pltpu.VMEM((_R, c), jnp.float32),
            pltpu.VMEM((c,), jnp.float32),
            pltpu.SemaphoreType.DMA,
            pltpu.SemaphoreType.DMA,
            pltpu.SemaphoreType.DMA,
        ],
    )(x)

    agg = pl.pallas_call(
        functools.partial(_sim_body, b=b, b0=b0, s_parts=_S, inv_n=1.0 / n),
        out_shape=jax.ShapeDtypeStruct((b, c), jnp.float32),
    )(pool_tc, part_sc)

    bb = 2
    out = pl.pallas_call(
        _add_body,
        grid=(b // bb,),
        in_specs=[
            pl.BlockSpec((bb, n, c), lambda i: (i, 0, 0)),
            pl.BlockSpec((bb, 1, c), lambda i: (i, 0, 0)),
        ],
        out_specs=pl.BlockSpec((bb, n, c), lambda i: (i, 0, 0)),
        out_shape=jax.ShapeDtypeStruct((b, n, c), jnp.float32),
    )(x, agg.reshape(b, 1, c))
    return out


# speedup vs baseline: 1.0235x; 1.0235x over previous
import functools

import jax
import jax.numpy as jnp
from jax import lax
from jax.experimental import pallas as pl
from jax.experimental.pallas import tpu as pltpu
from jax.experimental.pallas import tpu_sc as plsc

_NC = 2
_NS = 16
_L = 16

_B0 = 48
_S = 4
_R = 128


def _pool_tc_body(x_ref, o_ref, *, inv_n):
    o_ref[...] = jnp.sum(x_ref[...], axis=1, keepdims=True) * inv_n


def _sc_pool_body(x_hbm, out_hbm, buf0, buf1, buf2, stage, sem0, sem1, sem2, *,
                  b0, bsc, n, c, s_parts, r_chunk):
    cid = lax.axis_index("c")
    sid = lax.axis_index("s")
    per_core = bsc // _NC
    tasks_per_core = per_core * s_parts
    tpw = tasks_per_core // _NS
    rows_per_task = n // s_parts
    nchunks = rows_per_task // r_chunk
    bufs = (buf0, buf1, buf2)
    sems = (sem0, sem1, sem2)
    nlane = c // _L

    for t in range(tpw):
        task = sid * tpw + t
        slot = task // s_parts
        part = task % s_parts
        gbatch = b0 + cid * per_core + slot
        r0 = part * rows_per_task

        def _start(k):
            return pltpu.async_copy(
                x_hbm.at[gbatch, pl.ds(r0 + k * r_chunk, r_chunk), :],
                bufs[k % 3], sems[k % 3])

        unroll = 8

        def _mk_body(bref):
            def body(g, a):
                for dr in range(unroll):
                    a = tuple(a[l] + bref[g * unroll + dr, pl.ds(l * _L, _L)]
                              for l in range(nlane))
                return a
            return body

        acc = tuple(jnp.zeros((_L,), jnp.float32) for _ in range(nlane))
        cps = [None] * (nchunks + 1)
        cps[0] = _start(0)
        cps[1] = _start(1)
        for k in range(nchunks):
            if k + 2 < nchunks:
                cps[k + 2] = _start(k + 2)
            cps[k].wait()
            acc = lax.fori_loop(0, r_chunk // unroll,
                                _mk_body(bufs[k % 3]), acc)

        for l in range(nlane):
            stage[pl.ds(l * _L, _L)] = acc[l]
        gtask = cid * tasks_per_core + task
        pltpu.sync_copy(stage, out_hbm.at[gtask])


def _sim_body(ptc_ref, psc_ref, agg_ref, *, b, b0, s_parts, inv_n):
    bsc = b - b0
    p_tc = ptc_ref[...].reshape(b0, -1)
    part = psc_ref[...]
    g = bsc * s_parts
    per_core = bsc // _NC
    tasks_per_core = per_core * s_parts
    bi = lax.broadcasted_iota(jnp.int32, (bsc, g), 0)
    ti = lax.broadcasted_iota(jnp.int32, (bsc, g), 1)
    owner = (ti // tasks_per_core) * per_core + (ti % tasks_per_core) // s_parts
    m = jnp.where(owner == bi, 1.0, 0.0).astype(jnp.float32)
    p_sc = jnp.dot(m, part, preferred_element_type=jnp.float32) * inv_n
    p = jnp.concatenate([p_tc, p_sc], axis=0)

    s2 = jnp.sum(p * p, axis=1, keepdims=True)
    norm = jnp.sqrt(s2)
    xn = p / jnp.maximum(norm, 1e-12)
    s = jnp.dot(xn, xn.T, preferred_element_type=jnp.float32)
    rows = lax.broadcasted_iota(jnp.int32, (b, b), 0)
    cols = lax.broadcasted_iota(jnp.int32, (b, b), 1)
    s = jnp.where(rows == cols, 0.0, s)
    maxv = jnp.max(s, axis=1, keepdims=True)
    cand = jnp.where(s == maxv, cols, b)
    amin = jnp.min(cand, axis=1, keepdims=True)
    attn = jnp.where(cols == amin, maxv, 0.0)
    agg_ref[...] = jnp.dot(attn, p, preferred_element_type=jnp.float32)


def _add_body(x_ref, agg_ref, o_ref):
    o_ref[...] = x_ref[...] + agg_ref[...]


def kernel(x):
    b, n, c = x.shape
    b0 = _B0
    bsc = b - b0

    pb = 2
    pool_tc = pl.pallas_call(
        functools.partial(_pool_tc_body, inv_n=1.0 / n),
        grid=(b0 // pb,),
        in_specs=[pl.BlockSpec((pb, n, c), lambda i: (i, 0, 0))],
        out_specs=pl.BlockSpec((pb, 1, c), lambda i: (i, 0, 0)),
        out_shape=jax.ShapeDtypeStruct((b0, 1, c), jnp.float32),
    )(x)

    mesh = plsc.VectorSubcoreMesh(
        core_axis_name="c", subcore_axis_name="s",
        num_cores=_NC, num_subcores=_NS)
    part_sc = pl.kernel(
        functools.partial(_sc_pool_body, b0=b0, bsc=bsc, n=n, c=c,
                          s_parts=_S, r_chunk=_R),
        out_type=jax.ShapeDtypeStruct((bsc * _S, c), jnp.float32),
        mesh=mesh,
        scratch_types=[
            pltpu.VMEM((_R, c), jnp.float32),
            pltpu.VMEM((_R, c), jnp.float32),
            pltpu.VMEM((_R, c), jnp.float32),
            pltpu.VMEM((c,), jnp.float32),
            pltpu.SemaphoreType.DMA,
            pltpu.SemaphoreType.DMA,
            pltpu.SemaphoreType.DMA,
        ],
    )(x)

    agg = pl.pallas_call(
        functools.partial(_sim_body, b=b, b0=b0, s_parts=_S, inv_n=1.0 / n),
        out_shape=jax.ShapeDtypeStruct((b, c), jnp.float32),
    )(pool_tc, part_sc)

    bb = 2
    out = pl.pallas_call(
        _add_body,
        grid=(b // bb,),
        in_specs=[
            pl.BlockSpec((bb, n, c), lambda i: (i, 0, 0)),
            pl.BlockSpec((bb, 1, c), lambda i: (i, 0, 0)),
        ],
        out_specs=pl.BlockSpec((bb, n, c), lambda i: (i, 0, 0)),
        out_shape=jax.ShapeDtypeStruct((b, n, c), jnp.float32),
    )(x, agg.reshape(b, 1, c))
    return out


# speedup vs baseline: 1.0330x; 1.0093x over previous
import functools

import jax
import jax.numpy as jnp
from jax import lax
from jax.experimental import pallas as pl
from jax.experimental.pallas import tpu as pltpu
from jax.experimental.pallas import tpu_sc as plsc

_NC = 2
_NS = 16
_L = 16

_B0 = 60
_S = 8
_R = 128


def _pool_tc_body(x_ref, o_ref, *, inv_n):
    o_ref[...] = jnp.sum(x_ref[...], axis=1, keepdims=True) * inv_n


def _sc_pool_body(x_hbm, out_hbm, buf0, buf1, buf2, stage, sem0, sem1, sem2, *,
                  b0, bsc, n, c, s_parts, r_chunk):
    cid = lax.axis_index("c")
    sid = lax.axis_index("s")
    per_core = bsc // _NC
    tasks_per_core = per_core * s_parts
    tpw = tasks_per_core // _NS
    rows_per_task = n // s_parts
    nchunks = rows_per_task // r_chunk
    bufs = (buf0, buf1, buf2)
    sems = (sem0, sem1, sem2)
    nlane = c // _L

    for t in range(tpw):
        task = sid * tpw + t
        slot = task // s_parts
        part = task % s_parts
        gbatch = b0 + cid * per_core + slot
        r0 = part * rows_per_task

        def _start(k):
            return pltpu.async_copy(
                x_hbm.at[gbatch, pl.ds(r0 + k * r_chunk, r_chunk), :],
                bufs[k % 3], sems[k % 3])

        unroll = 8

        def _mk_body(bref):
            def body(g, a):
                for dr in range(unroll):
                    a = tuple(a[l] + bref[g * unroll + dr, pl.ds(l * _L, _L)]
                              for l in range(nlane))
                return a
            return body

        acc = tuple(jnp.zeros((_L,), jnp.float32) for _ in range(nlane))
        cps = [None] * (nchunks + 1)
        cps[0] = _start(0)
        cps[1] = _start(1)
        for k in range(nchunks):
            if k + 2 < nchunks:
                cps[k + 2] = _start(k + 2)
            cps[k].wait()
            acc = lax.fori_loop(0, r_chunk // unroll,
                                _mk_body(bufs[k % 3]), acc)

        for l in range(nlane):
            stage[pl.ds(l * _L, _L)] = acc[l]
        gtask = cid * tasks_per_core + task
        pltpu.sync_copy(stage, out_hbm.at[gtask])


def _sim_body(ptc_ref, psc_ref, agg_ref, *, b, b0, s_parts, inv_n):
    bsc = b - b0
    p_tc = ptc_ref[...].reshape(b0, -1)
    part = psc_ref[...]
    g = bsc * s_parts
    per_core = bsc // _NC
    tasks_per_core = per_core * s_parts
    bi = lax.broadcasted_iota(jnp.int32, (bsc, g), 0)
    ti = lax.broadcasted_iota(jnp.int32, (bsc, g), 1)
    owner = (ti // tasks_per_core) * per_core + (ti % tasks_per_core) // s_parts
    m = jnp.where(owner == bi, 1.0, 0.0).astype(jnp.float32)
    p_sc = jnp.dot(m, part, preferred_element_type=jnp.float32) * inv_n
    p = jnp.concatenate([p_tc, p_sc], axis=0)

    s2 = jnp.sum(p * p, axis=1, keepdims=True)
    norm = jnp.sqrt(s2)
    xn = p / jnp.maximum(norm, 1e-12)
    s = jnp.dot(xn, xn.T, preferred_element_type=jnp.float32)
    rows = lax.broadcasted_iota(jnp.int32, (b, b), 0)
    cols = lax.broadcasted_iota(jnp.int32, (b, b), 1)
    s = jnp.where(rows == cols, 0.0, s)
    maxv = jnp.max(s, axis=1, keepdims=True)
    cand = jnp.where(s == maxv, cols, b)
    amin = jnp.min(cand, axis=1, keepdims=True)
    attn = jnp.where(cols == amin, maxv, 0.0)
    agg_ref[...] = jnp.dot(attn, p, preferred_element_type=jnp.float32)


def _add_body(x_ref, agg_ref, o_ref):
    o_ref[...] = x_ref[...] + agg_ref[...]


def kernel(x):
    b, n, c = x.shape
    b0 = _B0
    bsc = b - b0

    mesh = plsc.VectorSubcoreMesh(
        core_axis_name="c", subcore_axis_name="s",
        num_cores=_NC, num_subcores=_NS)
    part_sc = pl.kernel(
        functools.partial(_sc_pool_body, b0=b0, bsc=bsc, n=n, c=c,
                          s_parts=_S, r_chunk=_R),
        out_type=jax.ShapeDtypeStruct((bsc * _S, c), jnp.float32),
        mesh=mesh,
        scratch_types=[
            pltpu.VMEM((_R, c), jnp.float32),
            pltpu.VMEM((_R, c), jnp.float32),
            pltpu.VMEM((_R, c), jnp.float32),
            pltpu.VMEM((c,), jnp.float32),
            pltpu.SemaphoreType.DMA,
            pltpu.SemaphoreType.DMA,
            pltpu.SemaphoreType.DMA,
        ],
    )(x)

    pb = 2
    pool_tc = pl.pallas_call(
        functools.partial(_pool_tc_body, inv_n=1.0 / n),
        grid=(b0 // pb,),
        in_specs=[pl.BlockSpec((pb, n, c), lambda i: (i, 0, 0))],
        out_specs=pl.BlockSpec((pb, 1, c), lambda i: (i, 0, 0)),
        out_shape=jax.ShapeDtypeStruct((b0, 1, c), jnp.float32),
    )(x)

    agg = pl.pallas_call(
        functools.partial(_sim_body, b=b, b0=b0, s_parts=_S, inv_n=1.0 / n),
        out_shape=jax.ShapeDtypeStruct((b, c), jnp.float32),
    )(pool_tc, part_sc)

    bb = 2
    out = pl.pallas_call(
        _add_body,
        grid=(b // bb,),
        in_specs=[
            pl.BlockSpec((bb, n, c), lambda i: (i, 0, 0)),
            pl.BlockSpec((bb, 1, c), lambda i: (i, 0, 0)),
        ],
        out_specs=pl.BlockSpec((bb, n, c), lambda i: (i, 0, 0)),
        out_shape=jax.ShapeDtypeStruct((b, n, c), jnp.float32),
    )(x, agg.reshape(b, 1, c))
    return out
